# baseline (speedup 1.0000x reference)
import functools
import math

import jax
import jax.numpy as jnp
from jax import lax
from jax.experimental import pallas as pl
from jax.experimental.pallas import tpu as pltpu
from jax.experimental.pallas import tpu_sc as plsc

D_MODEL = 64
SCALE = math.sqrt(D_MODEL)

_info = plsc.get_sparse_core_info()
NC = _info.num_cores
NS = _info.num_subcores
LANES = _info.num_lanes
NW = NC * NS

V = 1000000
N_POS = 50
N_SEQ = 16384
N_IC = N_SEQ // 128

_mesh = plsc.VectorSubcoreMesh(core_axis_name="c", subcore_axis_name="s")


@functools.partial(
    pl.kernel,
    mesh=_mesh,
    compiler_params=pltpu.CompilerParams(
        use_tc_tiling_on_sc=False, needs_layout_passes=False),
    out_type=jax.ShapeDtypeStruct((N_POS, 8, N_IC, 8, 128), jnp.float32),
    scratch_types=[
        pltpu.VMEM((128,), jnp.int32),
        pltpu.VMEM((128, D_MODEL), jnp.float32),
        pltpu.VMEM((8, 8, 128), jnp.float32),
        pltpu.SemaphoreType.DMA,
    ],
)
def _emb(x_t_hbm, table_hbm, out_hbm, xv, gbuf, outb, sem):
    wid = lax.axis_index("s") * NC + lax.axis_index("c")
    iota = lax.iota(jnp.int32, LANES)
    n_units = (N_POS * N_IC) // NW

    def unit(t, carry):
        u = t * NW + wid
        j = u // N_IC
        ic = u % N_IC
        pltpu.sync_copy(x_t_hbm.at[j, pl.ds(ic * 128, 128)], xv)
        pltpu.async_copy(table_hbm.at[xv], gbuf, sem).wait()

        def g_body(g, c1):
            l0 = g * LANES
            row = iota + l0

            def q_body(q, c2):
                col = jnp.full((LANES,), q, jnp.int32)
                vec = plsc.load_gather(gbuf, [row, col]) * SCALE
                outb[q // 8, q % 8, pl.ds(l0, LANES)] = vec
                return c2

            lax.fori_loop(0, D_MODEL, q_body, c1)
            return c1

        lax.fori_loop(0, 128 // LANES, g_body, 0)

        def w_body(r, c1):
            pltpu.sync_copy(outb.at[r], out_hbm.at[j, r, ic])
            return c1

        lax.fori_loop(0, 8, w_body, 0)
        return carry

    lax.fori_loop(0, n_units, unit, 0)


def kernel(x, lut):
    out5 = _emb(x.T, lut)
    out_t = out5.transpose(0, 1, 3, 2, 4).reshape(N_POS, D_MODEL, N_SEQ)
    return out_t.transpose(2, 0, 1)


# speedup vs baseline: 1.1578x; 1.1578x over previous
import functools
import math

import jax
import jax.numpy as jnp
from jax import lax
from jax.experimental import pallas as pl
from jax.experimental.pallas import tpu as pltpu
from jax.experimental.pallas import tpu_sc as plsc

D_MODEL = 64
SCALE = math.sqrt(D_MODEL)

_info = plsc.get_sparse_core_info()
NC = _info.num_cores
NS = _info.num_subcores
LANES = _info.num_lanes
NW = NC * NS

V = 1000000
N_POS = 50
N_SEQ = 16384
N_IC = N_SEQ // 128

_mesh = plsc.VectorSubcoreMesh(core_axis_name="c", subcore_axis_name="s")


@functools.partial(
    pl.kernel,
    mesh=_mesh,
    compiler_params=pltpu.CompilerParams(
        use_tc_tiling_on_sc=False, needs_layout_passes=False),
    out_type=jax.ShapeDtypeStruct((N_POS, 8, N_IC, 8, 128), jnp.float32),
    scratch_types=[
        pltpu.VMEM((2, 128), jnp.int32),
        pltpu.VMEM((2, 128, D_MODEL), jnp.float32),
        pltpu.VMEM((2, 8, 8, 128), jnp.float32),
        pltpu.SemaphoreType.DMA,
        pltpu.SemaphoreType.DMA,
        pltpu.SemaphoreType.DMA,
        pltpu.SemaphoreType.DMA,
        pltpu.SemaphoreType.DMA,
        pltpu.SemaphoreType.DMA,
    ],
)
def _emb(x_t_hbm, table_hbm, out_hbm, xv, gbuf, outb,
         xs0, xs1, gs0, gs1, os0, os1):
    wid = lax.axis_index("s") * NC + lax.axis_index("c")
    iota = lax.iota(jnp.int32, LANES)
    n_units = (N_POS * N_IC) // NW
    xsems = (xs0, xs1)
    gsems = (gs0, gs1)
    osems = (os0, os1)

    def ji(t):
        u = t * NW + wid
        return u // N_IC, u % N_IC

    def xv_copy(t, b):
        j, ic = ji(t)
        return pltpu.make_async_copy(
            x_t_hbm.at[j, pl.ds(ic * 128, 128)], xv.at[b], xsems[b])

    def gather(t, b):
        return pltpu.make_async_copy(
            table_hbm.at[xv.at[b]], gbuf.at[b], gsems[b])

    def out_copy(t, b):
        j, ic = ji(t)
        return pltpu.make_async_copy(
            outb.at[b], out_hbm.at[j, pl.ds(0, 8), ic], osems[b])

    def transpose(b):
        for g in range(128 // LANES):
            row = iota + g * LANES

            def q_body(qq, c2):
                for dq in range(8):
                    q = qq * 8 + dq
                    col = jnp.full((LANES,), q, jnp.int32)
                    vec = plsc.load_gather(gbuf.at[b], [row, col]) * SCALE
                    outb[b, q // 8, q % 8, pl.ds(g * LANES, LANES)] = vec
                return c2

            lax.fori_loop(0, 8, q_body, 0)

    xv_copy(0, 0).start()
    xv_copy(0, 0).wait()
    gather(0, 0).start()

    def unit(t, carry):
        b = lax.rem(t, 2)
        nb = 1 - b

        def body(b, nb):
            @pl.when(t + 1 < n_units)
            def _():
                xv_copy(t + 1, nb).start()

            gather(t, b).wait()

            @pl.when(t + 1 < n_units)
            def _():
                xv_copy(t + 1, nb).wait()

                @pl.when(t > 0)
                def _():
                    out_copy(t - 1, nb).wait()

                gather(t + 1, nb).start()

            transpose(b)
            out_copy(t, b).start()

        @pl.when(b == 0)
        def _():
            body(0, 1)

        @pl.when(b == 1)
        def _():
            body(1, 0)

        return carry

    lax.fori_loop(0, n_units, unit, 0)
    out_copy(n_units - 2, (n_units - 2) % 2).wait()
    out_copy(n_units - 1, (n_units - 1) % 2).wait()


def kernel(x, lut):
    out5 = _emb(x.T, lut)
    out_t = out5.transpose(0, 1, 3, 2, 4).reshape(N_POS, D_MODEL, N_SEQ)
    return out_t.transpose(2, 0, 1)


# speedup vs baseline: 1.9646x; 1.6968x over previous
import functools
import math

import jax
import jax.numpy as jnp
from jax import lax
from jax.experimental import pallas as pl
from jax.experimental.pallas import tpu as pltpu
from jax.experimental.pallas import tpu_sc as plsc

D_MODEL = 64
SCALE = math.sqrt(D_MODEL)

_info = plsc.get_sparse_core_info()
NC = _info.num_cores
NS = _info.num_subcores
LANES = _info.num_lanes
NW = NC * NS

V = 1000000
N_POS = 50
N_SEQ = 16384
N_IC = N_SEQ // 128

_mesh = plsc.VectorSubcoreMesh(core_axis_name="c", subcore_axis_name="s")


@functools.partial(
    pl.kernel,
    mesh=_mesh,
    compiler_params=pltpu.CompilerParams(
        use_tc_tiling_on_sc=False, needs_layout_passes=False),
    out_type=jax.ShapeDtypeStruct((N_POS, 8, N_IC, 8, 128), jnp.float32),
    scratch_types=[
        pltpu.VMEM((2, 128), jnp.int32),
        pltpu.VMEM((2, 128, D_MODEL), jnp.float32),
        pltpu.VMEM((2, 8, 8, 129), jnp.float32),
        pltpu.SemaphoreType.DMA,
        pltpu.SemaphoreType.DMA,
        pltpu.SemaphoreType.DMA,
        pltpu.SemaphoreType.DMA,
        pltpu.SemaphoreType.DMA,
        pltpu.SemaphoreType.DMA,
    ],
)
def _emb(x_t_hbm, table_hbm, out_hbm, xv, gbuf, outb,
         xs0, xs1, gs0, gs1, os0, os1):
    wid = lax.axis_index("s") * NC + lax.axis_index("c")
    iota = lax.iota(jnp.int32, LANES)
    n_units = (N_POS * N_IC) // NW
    xsems = (xs0, xs1)
    gsems = (gs0, gs1)
    osems = (os0, os1)

    def ji(t):
        u = t * NW + wid
        return u // N_IC, u % N_IC

    def xv_copy(t, b):
        j, ic = ji(t)
        return pltpu.make_async_copy(
            x_t_hbm.at[j, pl.ds(ic * 128, 128)], xv.at[b], xsems[b])

    def gather(t, b):
        return pltpu.make_async_copy(
            table_hbm.at[xv.at[b]], gbuf.at[b], gsems[b])

    def out_copy(t, b):
        j, ic = ji(t)
        return pltpu.make_async_copy(
            outb.at[b, pl.ds(0, 8), pl.ds(0, 8), pl.ds(0, 128)],
            out_hbm.at[j, pl.ds(0, 8), ic], osems[b])

    def transpose(b):
        qhi = [lax.shift_right_logical(iota + qg * LANES, 3)
               for qg in range(D_MODEL // LANES)]
        qlo = [lax.bitwise_and(iota + qg * LANES,
                               jnp.full((LANES,), 7, jnp.int32))
               for qg in range(D_MODEL // LANES)]

        def l_body(l, c2):
            col = jnp.full((LANES,), l, jnp.int32)
            for qg in range(D_MODEL // LANES):
                vec = gbuf[b, l, pl.ds(qg * LANES, LANES)] * SCALE
                plsc.store_scatter(
                    outb.at[b], [qhi[qg], qlo[qg], col], vec)
            return c2

        lax.fori_loop(0, 128, l_body, 0, unroll=8)

    xv_copy(0, 0).start()
    xv_copy(0, 0).wait()
    gather(0, 0).start()

    def unit(t, carry):
        b = lax.rem(t, 2)
        nb = 1 - b

        def body(b, nb):
            @pl.when(t + 1 < n_units)
            def _():
                xv_copy(t + 1, nb).start()

            gather(t, b).wait()

            @pl.when(t + 1 < n_units)
            def _():
                xv_copy(t + 1, nb).wait()

                @pl.when(t > 0)
                def _():
                    out_copy(t - 1, nb).wait()

                gather(t + 1, nb).start()

            transpose(b)
            out_copy(t, b).start()

        @pl.when(b == 0)
        def _():
            body(0, 1)

        @pl.when(b == 1)
        def _():
            body(1, 0)

        return carry

    lax.fori_loop(0, n_units, unit, 0)
    out_copy(n_units - 2, (n_units - 2) % 2).wait()
    out_copy(n_units - 1, (n_units - 1) % 2).wait()


def kernel(x, lut):
    out5 = _emb(x.T, lut)
    out_t = out5.transpose(0, 1, 3, 2, 4).reshape(N_POS, D_MODEL, N_SEQ)
    return out_t.transpose(2, 0, 1)


# speedup vs baseline: 2.1273x; 1.0829x over previous
import functools
import math

import jax
import jax.numpy as jnp
from jax import lax
from jax.experimental import pallas as pl
from jax.experimental.pallas import tpu as pltpu
from jax.experimental.pallas import tpu_sc as plsc

D_MODEL = 64
SCALE = math.sqrt(D_MODEL)

_info = plsc.get_sparse_core_info()
NC = _info.num_cores
NS = _info.num_subcores
LANES = _info.num_lanes
NW = NC * NS

V = 1000000
N_POS = 50
N_SEQ = 16384
N_IC = N_SEQ // 128
UIC = 2
UB = 128 * UIC
NBUF = 3

_mesh = plsc.VectorSubcoreMesh(core_axis_name="c", subcore_axis_name="s")


@functools.partial(
    pl.kernel,
    mesh=_mesh,
    compiler_params=pltpu.CompilerParams(
        use_tc_tiling_on_sc=False, needs_layout_passes=False),
    out_type=jax.ShapeDtypeStruct((N_POS, 8, N_IC, 8, 128), jnp.float32),
    scratch_types=[
        pltpu.VMEM((NBUF, UB), jnp.int32),
        pltpu.VMEM((NBUF, UB, D_MODEL), jnp.float32),
        pltpu.VMEM((NBUF, 8, UIC, 8, 129), jnp.float32),
        pltpu.SemaphoreType.DMA,
        pltpu.SemaphoreType.DMA,
        pltpu.SemaphoreType.DMA,
        pltpu.SemaphoreType.DMA,
        pltpu.SemaphoreType.DMA,
        pltpu.SemaphoreType.DMA,
        pltpu.SemaphoreType.DMA,
        pltpu.SemaphoreType.DMA,
        pltpu.SemaphoreType.DMA,
    ],
)
def _emb(x_t_hbm, table_hbm, out_hbm, xv, gbuf, outb,
         xs0, xs1, xs2, gs0, gs1, gs2, os0, os1, os2):
    wid = lax.axis_index("s") * NC + lax.axis_index("c")
    iota = lax.iota(jnp.int32, LANES)
    n_units = (N_POS * N_IC) // (UIC * NW)
    xsems = (xs0, xs1, xs2)
    gsems = (gs0, gs1, gs2)
    osems = (os0, os1, os2)

    def ji(t):
        u = t * NW + wid
        return u // (N_IC // UIC), (u % (N_IC // UIC)) * UIC

    def xv_copy(t, b):
        j, ic0 = ji(t)
        return pltpu.make_async_copy(
            x_t_hbm.at[j, pl.ds(ic0 * 128, UB)], xv.at[b], xsems[b])

    def gather(t, b):
        return pltpu.make_async_copy(
            table_hbm.at[xv.at[b]], gbuf.at[b], gsems[b])

    def out_copy(t, b):
        j, ic0 = ji(t)
        return pltpu.make_async_copy(
            outb.at[b, pl.ds(0, 8), pl.ds(0, UIC), pl.ds(0, 8),
                    pl.ds(0, 128)],
            out_hbm.at[j, pl.ds(0, 8), pl.ds(ic0, UIC)], osems[b])

    qhi = [lax.shift_right_logical(iota + qg * LANES, 3)
           for qg in range(D_MODEL // LANES)]
    qlo = [lax.bitwise_and(iota + qg * LANES,
                           jnp.full((LANES,), 7, jnp.int32))
           for qg in range(D_MODEL // LANES)]

    def transpose(b):
        def l_body(l, c2):
            ich = jnp.full((LANES,), l // 128, jnp.int32)
            col = jnp.full((LANES,), l % 128, jnp.int32)
            for qg in range(D_MODEL // LANES):
                vec = gbuf[b, l, pl.ds(qg * LANES, LANES)] * SCALE
                plsc.store_scatter(
                    outb.at[b], [qhi[qg], ich, qlo[qg], col], vec)
            return c2

        lax.fori_loop(0, UB, l_body, 0, unroll=8)

    xv_copy(0, 0).start()
    xv_copy(0, 0).wait()
    gather(0, 0).start()
    xv_copy(1, 1).start()

    def unit(t, carry):
        def body(b):
            nb = (b + 1) % NBUF
            nnb = (b + 2) % NBUF

            @pl.when(t + 2 < n_units)
            def _():
                xv_copy(t + 2, nnb).start()

            @pl.when(t + 1 < n_units)
            def _():
                xv_copy(t + 1, nb).wait()
                gather(t + 1, nb).start()

            @pl.when(t >= NBUF)
            def _():
                out_copy(t - NBUF, b).wait()

            gather(t, b).wait()
            transpose(b)
            out_copy(t, b).start()

        b = lax.rem(t, NBUF)
        for bb in range(NBUF):
            @pl.when(b == bb)
            def _():
                body(bb)

        return carry

    lax.fori_loop(0, n_units, unit, 0)
    for t in range(n_units - NBUF, n_units):
        out_copy(t, t % NBUF).wait()


def kernel(x, lut):
    out5 = _emb(x.T, lut)
    out_t = out5.transpose(0, 1, 3, 2, 4).reshape(N_POS, D_MODEL, N_SEQ)
    return out_t.transpose(2, 0, 1)
